# Initial kernel scaffold; baseline (speedup 1.0000x reference)
import jax
import jax.numpy as jnp
from jax.experimental import pallas as pl


def kernel(x, pos, batch, W, b, gamma, beta):
    raise NotImplementedError("write your pallas kernel here")


# speedup vs baseline: 11.1315x; 11.1315x over previous
import jax
import jax.numpy as jnp
from jax import lax
from jax.experimental import pallas as pl
from jax.experimental.pallas import tpu as pltpu
from jax.experimental.pallas import tpu_sc as plsc

_N = 16384
_NS = 4096
_K = 16
_IN_C = 128
_OUT_C = 256


def _fps_body(px_ref, py_ref, pz_ref, idx_ref, qx_ref, qy_ref, qz_ref):
    px = px_ref[...]
    py = py_ref[...]
    pz = pz_ref[...]
    iota_p = (lax.broadcasted_iota(jnp.int32, (128, 128), 0) * 128
              + lax.broadcasted_iota(jnp.int32, (128, 128), 1))
    iota_s = (lax.broadcasted_iota(jnp.int32, (32, 128), 0) * 128
              + lax.broadcasted_iota(jnp.int32, (32, 128), 1))
    big = jnp.int32(2 ** 30)

    def coords_at(m):
        sx = jnp.sum(jnp.where(m, px, 0.0))
        sy = jnp.sum(jnp.where(m, py, 0.0))
        sz = jnp.sum(jnp.where(m, pz, 0.0))
        return sx, sy, sz

    def dist(sx, sy, sz):
        dx = px - sx
        dy = py - sy
        dz = pz - sz
        return (dx * dx + dz * dz) + dy * dy

    sx0, sy0, sz0 = coords_at(iota_p == 0)
    min_d0 = dist(sx0, sy0, sz0)
    idxs0 = jnp.zeros((32, 128), jnp.int32)
    pm0 = iota_s == 0
    z = jnp.zeros((32, 128), jnp.float32)
    qx0 = jnp.where(pm0, sx0, z)
    qy0 = jnp.where(pm0, sy0, z)
    qz0 = jnp.where(pm0, sz0, z)

    def body(i, st):
        min_d, idxs, qx, qy, qz, sx, sy, sz = st
        min_d = jnp.minimum(min_d, dist(sx, sy, sz))
        mx = jnp.max(min_d)
        nxt = jnp.min(jnp.where(min_d == mx, iota_p, big))
        nsx, nsy, nsz = coords_at(iota_p == nxt)
        pm = iota_s == i
        idxs = jnp.where(pm, nxt, idxs)
        qx = jnp.where(pm, nsx, qx)
        qy = jnp.where(pm, nsy, qy)
        qz = jnp.where(pm, nsz, qz)
        return (min_d, idxs, qx, qy, qz, nsx, nsy, nsz)

    st = lax.fori_loop(1, _NS, body,
                       (min_d0, idxs0, qx0, qy0, qz0, sx0, sy0, sz0))
    _, idxs, qx, qy, qz, _, _, _ = st
    idx_ref[...] = idxs
    qx_ref[...] = qx
    qy_ref[...] = qy
    qz_ref[...] = qz


def _fps(px, py, pz):
    return pl.pallas_call(
        _fps_body,
        out_shape=[
            jax.ShapeDtypeStruct((32, 128), jnp.int32),
            jax.ShapeDtypeStruct((32, 128), jnp.float32),
            jax.ShapeDtypeStruct((32, 128), jnp.float32),
            jax.ShapeDtypeStruct((32, 128), jnp.float32),
        ],
    )(px, py, pz)


def _knn_body(q_ref, posp_ref, cn_ref, nbr_ref):
    q = q_ref[...]
    posp = posp_ref[...]
    cn = cn_ref[...]
    qn = jnp.sum(q * q, axis=1, keepdims=True)
    qc = lax.dot_general(q.astype(jnp.bfloat16), posp.astype(jnp.bfloat16),
                         (((1,), (1,)), ((), ())),
                         preferred_element_type=jnp.float32)
    d = (qn - 2.0 * qc) + cn
    iota_c = lax.broadcasted_iota(jnp.int32, (128, _N), 1)
    iota_k = lax.broadcasted_iota(jnp.int32, (128, _K), 1)
    big = jnp.int32(2 ** 30)
    inf = jnp.float32(jnp.inf)
    nbr = jnp.zeros((128, _K), jnp.int32)
    for k in range(_K):
        m = jnp.min(d, axis=1, keepdims=True)
        idx = jnp.min(jnp.where(d == m, iota_c, big), axis=1, keepdims=True)
        nbr = jnp.where(iota_k == k, idx, nbr)
        d = jnp.where(iota_c == idx, inf, d)
    nbr_ref[...] = nbr


def _knn(qpos, posp, cn):
    return pl.pallas_call(
        _knn_body,
        grid=(_NS // 128,),
        in_specs=[
            pl.BlockSpec((128, 8), lambda i: (i, 0)),
            pl.BlockSpec((_N, 8), lambda i: (0, 0)),
            pl.BlockSpec((1, _N), lambda i: (0, 0)),
        ],
        out_specs=pl.BlockSpec((128, _K), lambda i: (i, 0)),
        out_shape=jax.ShapeDtypeStruct((_NS, _K), jnp.int32),
    )(qpos, posp, cn)


def _mlp_a_body(x_ref, w_ref, b_ref, h_ref, st_ref, acc_ref):
    i = pl.program_id(0)

    @pl.when(i == 0)
    def _():
        acc_ref[...] = jnp.zeros_like(acc_ref)

    h = lax.dot_general(x_ref[...], w_ref[...], (((1,), (1,)), ((), ())),
                        preferred_element_type=jnp.float32) + b_ref[...]
    h_ref[...] = h
    acc_ref[0:1, :] = acc_ref[0:1, :] + jnp.sum(h, axis=0)[None]
    acc_ref[1:2, :] = acc_ref[1:2, :] + jnp.sum(h * h, axis=0)[None]

    @pl.when(i == pl.num_programs(0) - 1)
    def _():
        st_ref[...] = acc_ref[...]


def _mlp_b_body(h_ref, st_ref, g_ref, be_ref, o_ref):
    inv_n = jnp.float32(1.0 / _N)
    mean = st_ref[0:1, :] * inv_n
    var = st_ref[1:2, :] * inv_n - mean * mean
    inv = lax.rsqrt(var + 1e-5)
    t = (h_ref[...] - mean) * inv
    o_ref[...] = jnp.maximum(t * g_ref[...] + be_ref[...], 0.0)


def _mlp(x, w, b2, gamma2, beta2):
    nblk = 16
    rows = _N // nblk
    h, st = pl.pallas_call(
        _mlp_a_body,
        grid=(nblk,),
        in_specs=[
            pl.BlockSpec((rows, _IN_C), lambda i: (i, 0)),
            pl.BlockSpec((_OUT_C, _IN_C), lambda i: (0, 0)),
            pl.BlockSpec((1, _OUT_C), lambda i: (0, 0)),
        ],
        out_specs=[
            pl.BlockSpec((rows, _OUT_C), lambda i: (i, 0)),
            pl.BlockSpec((8, _OUT_C), lambda i: (0, 0)),
        ],
        out_shape=[
            jax.ShapeDtypeStruct((_N, _OUT_C), jnp.float32),
            jax.ShapeDtypeStruct((8, _OUT_C), jnp.float32),
        ],
        scratch_shapes=[pltpu.VMEM((8, _OUT_C), jnp.float32)],
    )(x, w, b2)
    return pl.pallas_call(
        _mlp_b_body,
        grid=(nblk,),
        in_specs=[
            pl.BlockSpec((rows, _OUT_C), lambda i: (i, 0)),
            pl.BlockSpec((8, _OUT_C), lambda i: (0, 0)),
            pl.BlockSpec((1, _OUT_C), lambda i: (0, 0)),
            pl.BlockSpec((1, _OUT_C), lambda i: (0, 0)),
        ],
        out_specs=pl.BlockSpec((rows, _OUT_C), lambda i: (i, 0)),
        out_shape=jax.ShapeDtypeStruct((_N, _OUT_C), jnp.float32),
    )(h, st, gamma2, beta2)


_NC = 2
_NSC = 16
_NW = _NC * _NSC
_QPW = _NS // _NW
_GQ = 8
_NG = _QPW // _GQ


def _segmax_body(h_hbm, nbr_hbm, batch_hbm, fidx_hbm, out_hbm, sb_hbm,
                 idx_v, rows_v, out_v, bidx_v, bval_v, sem):
    c = lax.axis_index("c")
    s = lax.axis_index("s")
    wid = s * _NC + c
    qbase = wid * _QPW

    pltpu.sync_copy(nbr_hbm.at[pl.ds(wid * _NG, _NG)], idx_v)

    pltpu.sync_copy(fidx_hbm.at[pl.ds(qbase, _QPW)], bidx_v)
    pltpu.async_copy(batch_hbm.at[bidx_v], bval_v, sem).wait()
    pltpu.sync_copy(bval_v, sb_hbm.at[pl.ds(qbase, _QPW)])

    def group(g, carry):
        pltpu.async_copy(h_hbm.at[idx_v.at[g]], rows_v, sem).wait()

        def per_q(q, carry2):
            r0 = q * _K
            for cc in range(_OUT_C // 16):
                sl = pl.ds(cc * 16, 16)
                acc = rows_v[r0, sl]
                for r in range(1, _K):
                    acc = jnp.maximum(acc, rows_v[r0 + r, sl])
                out_v[q, sl] = acc
            return carry2

        lax.fori_loop(0, _GQ, per_q, 0)
        pltpu.sync_copy(out_v, out_hbm.at[pl.ds(qbase + g * _GQ, _GQ)])
        return carry

    lax.fori_loop(0, _NG, group, 0)


def _segmax(h, nbr2, batch, fidx):
    mesh = plsc.VectorSubcoreMesh(core_axis_name="c", subcore_axis_name="s")
    f = pl.kernel(
        _segmax_body,
        out_type=[
            jax.ShapeDtypeStruct((_NS, _OUT_C), jnp.float32),
            jax.ShapeDtypeStruct((_NS,), jnp.int32),
        ],
        mesh=mesh,
        scratch_types=[
            pltpu.VMEM((_NG, 128), jnp.int32),
            pltpu.VMEM((_GQ * _K, _OUT_C), jnp.float32),
            pltpu.VMEM((_GQ, _OUT_C), jnp.float32),
            pltpu.VMEM((_QPW,), jnp.int32),
            pltpu.VMEM((_QPW,), jnp.int32),
            pltpu.SemaphoreType.DMA,
        ],
    )
    return f(h, nbr2, batch, fidx)


def kernel(x, pos, batch, W, b, gamma, beta):
    px = pos[:, 0].reshape(128, 128)
    py = pos[:, 1].reshape(128, 128)
    pz = pos[:, 2].reshape(128, 128)
    idxs2, qx2, qy2, qz2 = _fps(px, py, pz)

    sub_pos = jnp.stack(
        [qx2.reshape(-1), qy2.reshape(-1), qz2.reshape(-1)], axis=1)

    zq = jnp.zeros((_NS, 5), jnp.float32)
    qpos = jnp.concatenate(
        [qx2.reshape(-1, 1), qy2.reshape(-1, 1), qz2.reshape(-1, 1), zq],
        axis=1)
    posp = jnp.concatenate([pos, jnp.zeros((_N, 5), jnp.float32)], axis=1)
    cn = jnp.sum(pos * pos, axis=1).reshape(1, _N)
    nbr = _knn(qpos, posp, cn)

    h = _mlp(x, W, b.reshape(1, -1), gamma.reshape(1, -1),
             beta.reshape(1, -1))

    out, sub_batch = _segmax(h, nbr.reshape(_NS * _K // 128, 128), batch,
                             idxs2.reshape(-1))
    return (out, sub_pos, sub_batch)


# speedup vs baseline: 11.2769x; 1.0131x over previous
import jax
import jax.numpy as jnp
from jax import lax
from jax.experimental import pallas as pl
from jax.experimental.pallas import tpu as pltpu
from jax.experimental.pallas import tpu_sc as plsc

_N = 16384
_NS = 4096
_K = 16
_IN_C = 128
_OUT_C = 256


def _fps_body(px_ref, py_ref, pz_ref, idx_ref, qx_ref, qy_ref, qz_ref):
    px = px_ref[...]
    py = py_ref[...]
    pz = pz_ref[...]
    iota_p = (lax.broadcasted_iota(jnp.int32, (128, 128), 0) * 128
              + lax.broadcasted_iota(jnp.int32, (128, 128), 1))
    iota_s = (lax.broadcasted_iota(jnp.int32, (32, 128), 0) * 128
              + lax.broadcasted_iota(jnp.int32, (32, 128), 1))
    big = jnp.int32(2 ** 30)

    lane = lax.broadcasted_iota(jnp.int32, (1, 128), 1)

    def coords_at(nxt):
        r = nxt // 128
        cmask = lane == (nxt - r * 128)
        sx = jnp.sum(jnp.where(cmask, px_ref[pl.ds(r, 1), :], 0.0))
        sy = jnp.sum(jnp.where(cmask, py_ref[pl.ds(r, 1), :], 0.0))
        sz = jnp.sum(jnp.where(cmask, pz_ref[pl.ds(r, 1), :], 0.0))
        return sx, sy, sz

    def dist(sx, sy, sz):
        dx = px - sx
        dy = py - sy
        dz = pz - sz
        return (dx * dx + dz * dz) + dy * dy

    sx0, sy0, sz0 = coords_at(jnp.int32(0))
    min_d0 = dist(sx0, sy0, sz0)
    idxs0 = jnp.zeros((32, 128), jnp.int32)
    pm0 = iota_s == 0
    z = jnp.zeros((32, 128), jnp.float32)
    qx0 = jnp.where(pm0, sx0, z)
    qy0 = jnp.where(pm0, sy0, z)
    qz0 = jnp.where(pm0, sz0, z)

    def body(i, st):
        min_d, idxs, qx, qy, qz, sx, sy, sz = st
        min_d = jnp.minimum(min_d, dist(sx, sy, sz))
        mx = jnp.max(min_d)
        nxt = jnp.min(jnp.where(min_d == mx, iota_p, big))
        nsx, nsy, nsz = coords_at(nxt)
        pm = iota_s == i
        idxs = jnp.where(pm, nxt, idxs)
        qx = jnp.where(pm, nsx, qx)
        qy = jnp.where(pm, nsy, qy)
        qz = jnp.where(pm, nsz, qz)
        return (min_d, idxs, qx, qy, qz, nsx, nsy, nsz)

    st = lax.fori_loop(1, _NS, body,
                       (min_d0, idxs0, qx0, qy0, qz0, sx0, sy0, sz0))
    _, idxs, qx, qy, qz, _, _, _ = st
    idx_ref[...] = idxs
    qx_ref[...] = qx
    qy_ref[...] = qy
    qz_ref[...] = qz


def _fps(px, py, pz):
    return pl.pallas_call(
        _fps_body,
        out_shape=[
            jax.ShapeDtypeStruct((32, 128), jnp.int32),
            jax.ShapeDtypeStruct((32, 128), jnp.float32),
            jax.ShapeDtypeStruct((32, 128), jnp.float32),
            jax.ShapeDtypeStruct((32, 128), jnp.float32),
        ],
    )(px, py, pz)


def _knn_body(q_ref, posp_ref, cn_ref, nbr_ref):
    q = q_ref[...]
    posp = posp_ref[...]
    cn = cn_ref[...]
    qn = jnp.sum(q * q, axis=1, keepdims=True)
    qc = lax.dot_general(q.astype(jnp.bfloat16), posp.astype(jnp.bfloat16),
                         (((1,), (1,)), ((), ())),
                         preferred_element_type=jnp.float32)
    d = (qn - 2.0 * qc) + cn
    iota_c = lax.broadcasted_iota(jnp.int32, (128, _N), 1)
    iota_k = lax.broadcasted_iota(jnp.int32, (128, _K), 1)
    big = jnp.int32(2 ** 30)
    inf = jnp.float32(jnp.inf)
    nbr = jnp.zeros((128, _K), jnp.int32)
    m = jnp.min(d, axis=1, keepdims=True)
    for k in range(_K):
        idx = jnp.min(jnp.where(d == m, iota_c, big), axis=1, keepdims=True)
        nbr = jnp.where(iota_k == k, idx, nbr)
        if k + 1 < _K:
            d = jnp.where(iota_c == idx, inf, d)
            m = jnp.min(d, axis=1, keepdims=True)
    nbr_ref[...] = nbr


def _knn(qpos, posp, cn):
    return pl.pallas_call(
        _knn_body,
        grid=(_NS // 128,),
        in_specs=[
            pl.BlockSpec((128, 8), lambda i: (i, 0)),
            pl.BlockSpec((_N, 8), lambda i: (0, 0)),
            pl.BlockSpec((1, _N), lambda i: (0, 0)),
        ],
        out_specs=pl.BlockSpec((128, _K), lambda i: (i, 0)),
        out_shape=jax.ShapeDtypeStruct((_NS, _K), jnp.int32),
    )(qpos, posp, cn)


def _mlp_a_body(x_ref, w_ref, b_ref, h_ref, st_ref, acc_ref):
    i = pl.program_id(0)

    @pl.when(i == 0)
    def _():
        acc_ref[...] = jnp.zeros_like(acc_ref)

    h = lax.dot_general(x_ref[...], w_ref[...], (((1,), (1,)), ((), ())),
                        preferred_element_type=jnp.float32) + b_ref[...]
    h_ref[...] = h
    acc_ref[0:1, :] = acc_ref[0:1, :] + jnp.sum(h, axis=0)[None]
    acc_ref[1:2, :] = acc_ref[1:2, :] + jnp.sum(h * h, axis=0)[None]

    @pl.when(i == pl.num_programs(0) - 1)
    def _():
        st_ref[...] = acc_ref[...]


def _mlp_b_body(h_ref, st_ref, g_ref, be_ref, o_ref):
    inv_n = jnp.float32(1.0 / _N)
    mean = st_ref[0:1, :] * inv_n
    var = st_ref[1:2, :] * inv_n - mean * mean
    inv = lax.rsqrt(var + 1e-5)
    t = (h_ref[...] - mean) * inv
    o_ref[...] = jnp.maximum(t * g_ref[...] + be_ref[...], 0.0)


def _mlp(x, w, b2, gamma2, beta2):
    nblk = 16
    rows = _N // nblk
    h, st = pl.pallas_call(
        _mlp_a_body,
        grid=(nblk,),
        in_specs=[
            pl.BlockSpec((rows, _IN_C), lambda i: (i, 0)),
            pl.BlockSpec((_OUT_C, _IN_C), lambda i: (0, 0)),
            pl.BlockSpec((1, _OUT_C), lambda i: (0, 0)),
        ],
        out_specs=[
            pl.BlockSpec((rows, _OUT_C), lambda i: (i, 0)),
            pl.BlockSpec((8, _OUT_C), lambda i: (0, 0)),
        ],
        out_shape=[
            jax.ShapeDtypeStruct((_N, _OUT_C), jnp.float32),
            jax.ShapeDtypeStruct((8, _OUT_C), jnp.float32),
        ],
        scratch_shapes=[pltpu.VMEM((8, _OUT_C), jnp.float32)],
    )(x, w, b2)
    return pl.pallas_call(
        _mlp_b_body,
        grid=(nblk,),
        in_specs=[
            pl.BlockSpec((rows, _OUT_C), lambda i: (i, 0)),
            pl.BlockSpec((8, _OUT_C), lambda i: (0, 0)),
            pl.BlockSpec((1, _OUT_C), lambda i: (0, 0)),
            pl.BlockSpec((1, _OUT_C), lambda i: (0, 0)),
        ],
        out_specs=pl.BlockSpec((rows, _OUT_C), lambda i: (i, 0)),
        out_shape=jax.ShapeDtypeStruct((_N, _OUT_C), jnp.float32),
    )(h, st, gamma2, beta2)


_NC = 2
_NSC = 16
_NW = _NC * _NSC
_QPW = _NS // _NW
_GQ = 8
_NG = _QPW // _GQ


def _segmax_body(h_hbm, nbr_hbm, batch_hbm, fidx_hbm, out_hbm, sb_hbm,
                 idx_v, rows_v, out_v, bidx_v, bval_v, sem):
    c = lax.axis_index("c")
    s = lax.axis_index("s")
    wid = s * _NC + c
    qbase = wid * _QPW

    pltpu.sync_copy(nbr_hbm.at[pl.ds(wid * _NG, _NG)], idx_v)

    pltpu.sync_copy(fidx_hbm.at[pl.ds(qbase, _QPW)], bidx_v)
    pltpu.async_copy(batch_hbm.at[bidx_v], bval_v, sem).wait()
    pltpu.sync_copy(bval_v, sb_hbm.at[pl.ds(qbase, _QPW)])

    def group(g, carry):
        pltpu.async_copy(h_hbm.at[idx_v.at[g]], rows_v, sem).wait()

        def per_q(q, carry2):
            r0 = q * _K
            for cc in range(_OUT_C // 16):
                sl = pl.ds(cc * 16, 16)
                acc = rows_v[r0, sl]
                for r in range(1, _K):
                    acc = jnp.maximum(acc, rows_v[r0 + r, sl])
                out_v[q, sl] = acc
            return carry2

        lax.fori_loop(0, _GQ, per_q, 0)
        pltpu.sync_copy(out_v, out_hbm.at[pl.ds(qbase + g * _GQ, _GQ)])
        return carry

    lax.fori_loop(0, _NG, group, 0)


def _segmax(h, nbr2, batch, fidx):
    mesh = plsc.VectorSubcoreMesh(core_axis_name="c", subcore_axis_name="s")
    f = pl.kernel(
        _segmax_body,
        out_type=[
            jax.ShapeDtypeStruct((_NS, _OUT_C), jnp.float32),
            jax.ShapeDtypeStruct((_NS,), jnp.int32),
        ],
        mesh=mesh,
        scratch_types=[
            pltpu.VMEM((_NG, 128), jnp.int32),
            pltpu.VMEM((_GQ * _K, _OUT_C), jnp.float32),
            pltpu.VMEM((_GQ, _OUT_C), jnp.float32),
            pltpu.VMEM((_QPW,), jnp.int32),
            pltpu.VMEM((_QPW,), jnp.int32),
            pltpu.SemaphoreType.DMA,
        ],
    )
    return f(h, nbr2, batch, fidx)


def kernel(x, pos, batch, W, b, gamma, beta):
    px = pos[:, 0].reshape(128, 128)
    py = pos[:, 1].reshape(128, 128)
    pz = pos[:, 2].reshape(128, 128)
    idxs2, qx2, qy2, qz2 = _fps(px, py, pz)

    sub_pos = jnp.stack(
        [qx2.reshape(-1), qy2.reshape(-1), qz2.reshape(-1)], axis=1)

    zq = jnp.zeros((_NS, 5), jnp.float32)
    qpos = jnp.concatenate(
        [qx2.reshape(-1, 1), qy2.reshape(-1, 1), qz2.reshape(-1, 1), zq],
        axis=1)
    posp = jnp.concatenate([pos, jnp.zeros((_N, 5), jnp.float32)], axis=1)
    cn = jnp.sum(pos * pos, axis=1).reshape(1, _N)
    nbr = _knn(qpos, posp, cn)

    h = _mlp(x, W, b.reshape(1, -1), gamma.reshape(1, -1),
             beta.reshape(1, -1))

    out, sub_batch = _segmax(h, nbr.reshape(_NS * _K // 128, 128), batch,
                             idxs2.reshape(-1))
    return (out, sub_pos, sub_batch)


# speedup vs baseline: 11.4560x; 1.0159x over previous
import jax
import jax.numpy as jnp
from jax import lax
from jax.experimental import pallas as pl
from jax.experimental.pallas import tpu as pltpu
from jax.experimental.pallas import tpu_sc as plsc

_N = 16384
_NS = 4096
_K = 16
_IN_C = 128
_OUT_C = 256


def _fps_body(px_ref, py_ref, pz_ref, idx_ref, qx_ref, qy_ref, qz_ref):
    px = px_ref[...]
    py = py_ref[...]
    pz = pz_ref[...]
    iota_p = (lax.broadcasted_iota(jnp.int32, (128, 128), 0) * 128
              + lax.broadcasted_iota(jnp.int32, (128, 128), 1))
    iota_s = (lax.broadcasted_iota(jnp.int32, (32, 128), 0) * 128
              + lax.broadcasted_iota(jnp.int32, (32, 128), 1))
    big = jnp.int32(2 ** 30)

    lane = lax.broadcasted_iota(jnp.int32, (1, 128), 1)

    def coords_at(nxt):
        r = nxt // 128
        cmask = lane == (nxt - r * 128)
        sx = jnp.sum(jnp.where(cmask, px_ref[pl.ds(r, 1), :], 0.0))
        sy = jnp.sum(jnp.where(cmask, py_ref[pl.ds(r, 1), :], 0.0))
        sz = jnp.sum(jnp.where(cmask, pz_ref[pl.ds(r, 1), :], 0.0))
        return sx, sy, sz

    def dist(sx, sy, sz):
        dx = px - sx
        dy = py - sy
        dz = pz - sz
        return (dx * dx + dz * dz) + dy * dy

    sx0, sy0, sz0 = coords_at(jnp.int32(0))
    min_d0 = dist(sx0, sy0, sz0)
    idxs0 = jnp.zeros((32, 128), jnp.int32)
    pm0 = iota_s == 0
    z = jnp.zeros((32, 128), jnp.float32)
    qx0 = jnp.where(pm0, sx0, z)
    qy0 = jnp.where(pm0, sy0, z)
    qz0 = jnp.where(pm0, sz0, z)

    def body(i, st):
        min_d, idxs, qx, qy, qz, sx, sy, sz = st
        min_d = jnp.minimum(min_d, dist(sx, sy, sz))
        mx = jnp.max(min_d)
        nxt = jnp.min(jnp.where(min_d == mx, iota_p, big))
        nsx, nsy, nsz = coords_at(nxt)
        pm = iota_s == i
        idxs = jnp.where(pm, nxt, idxs)
        qx = jnp.where(pm, nsx, qx)
        qy = jnp.where(pm, nsy, qy)
        qz = jnp.where(pm, nsz, qz)
        return (min_d, idxs, qx, qy, qz, nsx, nsy, nsz)

    st = lax.fori_loop(1, _NS, body,
                       (min_d0, idxs0, qx0, qy0, qz0, sx0, sy0, sz0))
    _, idxs, qx, qy, qz, _, _, _ = st
    idx_ref[...] = idxs
    qx_ref[...] = qx
    qy_ref[...] = qy
    qz_ref[...] = qz


def _fps(px, py, pz):
    return pl.pallas_call(
        _fps_body,
        out_shape=[
            jax.ShapeDtypeStruct((32, 128), jnp.int32),
            jax.ShapeDtypeStruct((32, 128), jnp.float32),
            jax.ShapeDtypeStruct((32, 128), jnp.float32),
            jax.ShapeDtypeStruct((32, 128), jnp.float32),
        ],
    )(px, py, pz)


def _knn_body(q_ref, posp_ref, cn_ref, nbr_ref):
    q = q_ref[...]
    posp = posp_ref[...]
    cn = cn_ref[...]
    qn = jnp.sum(q * q, axis=1, keepdims=True)
    qc = lax.dot_general(q.astype(jnp.bfloat16), posp.astype(jnp.bfloat16),
                         (((1,), (1,)), ((), ())),
                         preferred_element_type=jnp.float32)
    d = (qn - 2.0 * qc) + cn
    iota_c = lax.broadcasted_iota(jnp.int32, (128, _N), 1)
    iota_k = lax.broadcasted_iota(jnp.int32, (128, _K), 1)
    big = jnp.int32(2 ** 30)
    inf = jnp.float32(jnp.inf)
    nbr = jnp.zeros((128, _K), jnp.int32)
    for k in range(_K - 1):
        idx = jnp.argmin(d, axis=1).astype(jnp.int32)[:, None]
        nbr = jnp.where(iota_k == k, idx, nbr)
        d = jnp.where(iota_c == idx, inf, d)
    m = jnp.min(d, axis=1, keepdims=True)
    idx = jnp.min(jnp.where(d == m, iota_c, big), axis=1, keepdims=True)
    nbr = jnp.where(iota_k == _K - 1, idx, nbr)
    nbr_ref[...] = nbr


def _knn(qpos, posp, cn):
    return pl.pallas_call(
        _knn_body,
        grid=(_NS // 128,),
        in_specs=[
            pl.BlockSpec((128, 8), lambda i: (i, 0)),
            pl.BlockSpec((_N, 8), lambda i: (0, 0)),
            pl.BlockSpec((1, _N), lambda i: (0, 0)),
        ],
        out_specs=pl.BlockSpec((128, _K), lambda i: (i, 0)),
        out_shape=jax.ShapeDtypeStruct((_NS, _K), jnp.int32),
    )(qpos, posp, cn)


def _mlp_a_body(x_ref, w_ref, b_ref, h_ref, st_ref, acc_ref):
    i = pl.program_id(0)

    @pl.when(i == 0)
    def _():
        acc_ref[...] = jnp.zeros_like(acc_ref)

    h = lax.dot_general(x_ref[...], w_ref[...], (((1,), (1,)), ((), ())),
                        preferred_element_type=jnp.float32) + b_ref[...]
    h_ref[...] = h
    acc_ref[0:1, :] = acc_ref[0:1, :] + jnp.sum(h, axis=0)[None]
    acc_ref[1:2, :] = acc_ref[1:2, :] + jnp.sum(h * h, axis=0)[None]

    @pl.when(i == pl.num_programs(0) - 1)
    def _():
        st_ref[...] = acc_ref[...]


def _mlp_b_body(h_ref, st_ref, g_ref, be_ref, o_ref):
    inv_n = jnp.float32(1.0 / _N)
    mean = st_ref[0:1, :] * inv_n
    var = st_ref[1:2, :] * inv_n - mean * mean
    inv = lax.rsqrt(var + 1e-5)
    t = (h_ref[...] - mean) * inv
    o_ref[...] = jnp.maximum(t * g_ref[...] + be_ref[...], 0.0)


def _mlp(x, w, b2, gamma2, beta2):
    nblk = 16
    rows = _N // nblk
    h, st = pl.pallas_call(
        _mlp_a_body,
        grid=(nblk,),
        in_specs=[
            pl.BlockSpec((rows, _IN_C), lambda i: (i, 0)),
            pl.BlockSpec((_OUT_C, _IN_C), lambda i: (0, 0)),
            pl.BlockSpec((1, _OUT_C), lambda i: (0, 0)),
        ],
        out_specs=[
            pl.BlockSpec((rows, _OUT_C), lambda i: (i, 0)),
            pl.BlockSpec((8, _OUT_C), lambda i: (0, 0)),
        ],
        out_shape=[
            jax.ShapeDtypeStruct((_N, _OUT_C), jnp.float32),
            jax.ShapeDtypeStruct((8, _OUT_C), jnp.float32),
        ],
        scratch_shapes=[pltpu.VMEM((8, _OUT_C), jnp.float32)],
    )(x, w, b2)
    return pl.pallas_call(
        _mlp_b_body,
        grid=(nblk,),
        in_specs=[
            pl.BlockSpec((rows, _OUT_C), lambda i: (i, 0)),
            pl.BlockSpec((8, _OUT_C), lambda i: (0, 0)),
            pl.BlockSpec((1, _OUT_C), lambda i: (0, 0)),
            pl.BlockSpec((1, _OUT_C), lambda i: (0, 0)),
        ],
        out_specs=pl.BlockSpec((rows, _OUT_C), lambda i: (i, 0)),
        out_shape=jax.ShapeDtypeStruct((_N, _OUT_C), jnp.float32),
    )(h, st, gamma2, beta2)


_NC = 2
_NSC = 16
_NW = _NC * _NSC
_QPW = _NS // _NW
_GQ = 8
_NG = _QPW // _GQ


def _segmax_body(h_hbm, nbr_hbm, batch_hbm, fidx_hbm, out_hbm, sb_hbm,
                 idx_v, rows_v, out_v, bidx_v, bval_v, sem):
    c = lax.axis_index("c")
    s = lax.axis_index("s")
    wid = s * _NC + c
    qbase = wid * _QPW

    pltpu.sync_copy(nbr_hbm.at[pl.ds(wid * _NG, _NG)], idx_v)

    pltpu.sync_copy(fidx_hbm.at[pl.ds(qbase, _QPW)], bidx_v)
    pltpu.async_copy(batch_hbm.at[bidx_v], bval_v, sem).wait()
    pltpu.sync_copy(bval_v, sb_hbm.at[pl.ds(qbase, _QPW)])

    def group(g, carry):
        pltpu.async_copy(h_hbm.at[idx_v.at[g]], rows_v, sem).wait()

        def per_q(q, carry2):
            r0 = q * _K
            for cc in range(_OUT_C // 16):
                sl = pl.ds(cc * 16, 16)
                acc = rows_v[r0, sl]
                for r in range(1, _K):
                    acc = jnp.maximum(acc, rows_v[r0 + r, sl])
                out_v[q, sl] = acc
            return carry2

        lax.fori_loop(0, _GQ, per_q, 0)
        pltpu.sync_copy(out_v, out_hbm.at[pl.ds(qbase + g * _GQ, _GQ)])
        return carry

    lax.fori_loop(0, _NG, group, 0)


def _segmax(h, nbr2, batch, fidx):
    mesh = plsc.VectorSubcoreMesh(core_axis_name="c", subcore_axis_name="s")
    f = pl.kernel(
        _segmax_body,
        out_type=[
            jax.ShapeDtypeStruct((_NS, _OUT_C), jnp.float32),
            jax.ShapeDtypeStruct((_NS,), jnp.int32),
        ],
        mesh=mesh,
        scratch_types=[
            pltpu.VMEM((_NG, 128), jnp.int32),
            pltpu.VMEM((_GQ * _K, _OUT_C), jnp.float32),
            pltpu.VMEM((_GQ, _OUT_C), jnp.float32),
            pltpu.VMEM((_QPW,), jnp.int32),
            pltpu.VMEM((_QPW,), jnp.int32),
            pltpu.SemaphoreType.DMA,
        ],
    )
    return f(h, nbr2, batch, fidx)


def kernel(x, pos, batch, W, b, gamma, beta):
    px = pos[:, 0].reshape(128, 128)
    py = pos[:, 1].reshape(128, 128)
    pz = pos[:, 2].reshape(128, 128)
    idxs2, qx2, qy2, qz2 = _fps(px, py, pz)

    sub_pos = jnp.stack(
        [qx2.reshape(-1), qy2.reshape(-1), qz2.reshape(-1)], axis=1)

    zq = jnp.zeros((_NS, 5), jnp.float32)
    qpos = jnp.concatenate(
        [qx2.reshape(-1, 1), qy2.reshape(-1, 1), qz2.reshape(-1, 1), zq],
        axis=1)
    posp = jnp.concatenate([pos, jnp.zeros((_N, 5), jnp.float32)], axis=1)
    cn = jnp.sum(pos * pos, axis=1).reshape(1, _N)
    nbr = _knn(qpos, posp, cn)

    h = _mlp(x, W, b.reshape(1, -1), gamma.reshape(1, -1),
             beta.reshape(1, -1))

    out, sub_batch = _segmax(h, nbr.reshape(_NS * _K // 128, 128), batch,
                             idxs2.reshape(-1))
    return (out, sub_pos, sub_batch)


# speedup vs baseline: 11.4736x; 1.0015x over previous
import jax
import jax.numpy as jnp
from jax import lax
from jax.experimental import pallas as pl
from jax.experimental.pallas import tpu as pltpu
from jax.experimental.pallas import tpu_sc as plsc

_N = 16384
_NS = 4096
_K = 16
_IN_C = 128
_OUT_C = 256


def _fps_body(px_ref, py_ref, pz_ref, idx_ref, qx_ref, qy_ref, qz_ref):
    px = px_ref[...]
    py = py_ref[...]
    pz = pz_ref[...]
    iota_p = (lax.broadcasted_iota(jnp.int32, (128, 128), 0) * 128
              + lax.broadcasted_iota(jnp.int32, (128, 128), 1))
    iota_s = (lax.broadcasted_iota(jnp.int32, (32, 128), 0) * 128
              + lax.broadcasted_iota(jnp.int32, (32, 128), 1))
    big = jnp.int32(2 ** 30)

    lane = lax.broadcasted_iota(jnp.int32, (1, 128), 1)

    def coords_at(nxt):
        r = nxt // 128
        cmask = lane == (nxt - r * 128)
        sx = jnp.sum(jnp.where(cmask, px_ref[pl.ds(r, 1), :], 0.0),
                     keepdims=True)
        sy = jnp.sum(jnp.where(cmask, py_ref[pl.ds(r, 1), :], 0.0),
                     keepdims=True)
        sz = jnp.sum(jnp.where(cmask, pz_ref[pl.ds(r, 1), :], 0.0),
                     keepdims=True)
        return sx, sy, sz

    def dist(sx, sy, sz):
        dx = px - sx
        dy = py - sy
        dz = pz - sz
        return (dx * dx + dz * dz) + dy * dy

    sx0, sy0, sz0 = coords_at(jnp.int32(0))
    min_d0 = dist(sx0, sy0, sz0)
    idxs0 = jnp.zeros((32, 128), jnp.int32)
    pm0 = iota_s == 0
    z = jnp.zeros((32, 128), jnp.float32)
    qx0 = jnp.where(pm0, sx0, z)
    qy0 = jnp.where(pm0, sy0, z)
    qz0 = jnp.where(pm0, sz0, z)

    def body(i, st):
        min_d, idxs, qx, qy, qz, sx, sy, sz = st
        min_d = jnp.minimum(min_d, dist(sx, sy, sz))
        mx = jnp.max(min_d, keepdims=True)
        nxt = jnp.min(jnp.where(min_d == mx, iota_p, big))
        nsx, nsy, nsz = coords_at(nxt)
        pm = iota_s == i
        idxs = jnp.where(pm, nxt, idxs)
        qx = jnp.where(pm, nsx, qx)
        qy = jnp.where(pm, nsy, qy)
        qz = jnp.where(pm, nsz, qz)
        return (min_d, idxs, qx, qy, qz, nsx, nsy, nsz)

    st = lax.fori_loop(1, _NS, body,
                       (min_d0, idxs0, qx0, qy0, qz0, sx0, sy0, sz0))
    _, idxs, qx, qy, qz, _, _, _ = st
    idx_ref[...] = idxs
    qx_ref[...] = qx
    qy_ref[...] = qy
    qz_ref[...] = qz


def _fps(px, py, pz):
    return pl.pallas_call(
        _fps_body,
        out_shape=[
            jax.ShapeDtypeStruct((32, 128), jnp.int32),
            jax.ShapeDtypeStruct((32, 128), jnp.float32),
            jax.ShapeDtypeStruct((32, 128), jnp.float32),
            jax.ShapeDtypeStruct((32, 128), jnp.float32),
        ],
    )(px, py, pz)


def _knn_body(q_ref, posp_ref, cn_ref, nbr_ref):
    q = q_ref[...]
    posp = posp_ref[...]
    cn = cn_ref[...]
    qn = jnp.sum(q * q, axis=1, keepdims=True)
    qc = lax.dot_general(q.astype(jnp.bfloat16), posp.astype(jnp.bfloat16),
                         (((1,), (1,)), ((), ())),
                         preferred_element_type=jnp.float32)
    d = (qn - 2.0 * qc) + cn
    iota_c = lax.broadcasted_iota(jnp.int32, (128, _N), 1)
    iota_k = lax.broadcasted_iota(jnp.int32, (128, _K), 1)
    big = jnp.int32(2 ** 30)
    inf = jnp.float32(jnp.inf)
    nbr = jnp.zeros((128, _K), jnp.int32)
    for k in range(_K - 1):
        idx = jnp.argmin(d, axis=1).astype(jnp.int32)[:, None]
        nbr = jnp.where(iota_k == k, idx, nbr)
        d = jnp.where(iota_c == idx, inf, d)
    m = jnp.min(d, axis=1, keepdims=True)
    idx = jnp.min(jnp.where(d == m, iota_c, big), axis=1, keepdims=True)
    nbr = jnp.where(iota_k == _K - 1, idx, nbr)
    nbr_ref[...] = nbr


def _knn(qpos, posp, cn):
    return pl.pallas_call(
        _knn_body,
        grid=(_NS // 128,),
        in_specs=[
            pl.BlockSpec((128, 8), lambda i: (i, 0)),
            pl.BlockSpec((_N, 8), lambda i: (0, 0)),
            pl.BlockSpec((1, _N), lambda i: (0, 0)),
        ],
        out_specs=pl.BlockSpec((128, _K), lambda i: (i, 0)),
        out_shape=jax.ShapeDtypeStruct((_NS, _K), jnp.int32),
    )(qpos, posp, cn)


def _mlp_a_body(x_ref, w_ref, b_ref, h_ref, st_ref, acc_ref):
    i = pl.program_id(0)

    @pl.when(i == 0)
    def _():
        acc_ref[...] = jnp.zeros_like(acc_ref)

    h = lax.dot_general(x_ref[...], w_ref[...], (((1,), (1,)), ((), ())),
                        preferred_element_type=jnp.float32) + b_ref[...]
    h_ref[...] = h
    acc_ref[0:1, :] = acc_ref[0:1, :] + jnp.sum(h, axis=0)[None]
    acc_ref[1:2, :] = acc_ref[1:2, :] + jnp.sum(h * h, axis=0)[None]

    @pl.when(i == pl.num_programs(0) - 1)
    def _():
        st_ref[...] = acc_ref[...]


def _mlp_b_body(h_ref, st_ref, g_ref, be_ref, o_ref):
    inv_n = jnp.float32(1.0 / _N)
    mean = st_ref[0:1, :] * inv_n
    var = st_ref[1:2, :] * inv_n - mean * mean
    inv = lax.rsqrt(var + 1e-5)
    t = (h_ref[...] - mean) * inv
    o_ref[...] = jnp.maximum(t * g_ref[...] + be_ref[...], 0.0)


def _mlp(x, w, b2, gamma2, beta2):
    nblk = 16
    rows = _N // nblk
    h, st = pl.pallas_call(
        _mlp_a_body,
        grid=(nblk,),
        in_specs=[
            pl.BlockSpec((rows, _IN_C), lambda i: (i, 0)),
            pl.BlockSpec((_OUT_C, _IN_C), lambda i: (0, 0)),
            pl.BlockSpec((1, _OUT_C), lambda i: (0, 0)),
        ],
        out_specs=[
            pl.BlockSpec((rows, _OUT_C), lambda i: (i, 0)),
            pl.BlockSpec((8, _OUT_C), lambda i: (0, 0)),
        ],
        out_shape=[
            jax.ShapeDtypeStruct((_N, _OUT_C), jnp.float32),
            jax.ShapeDtypeStruct((8, _OUT_C), jnp.float32),
        ],
        scratch_shapes=[pltpu.VMEM((8, _OUT_C), jnp.float32)],
    )(x, w, b2)
    return pl.pallas_call(
        _mlp_b_body,
        grid=(nblk,),
        in_specs=[
            pl.BlockSpec((rows, _OUT_C), lambda i: (i, 0)),
            pl.BlockSpec((8, _OUT_C), lambda i: (0, 0)),
            pl.BlockSpec((1, _OUT_C), lambda i: (0, 0)),
            pl.BlockSpec((1, _OUT_C), lambda i: (0, 0)),
        ],
        out_specs=pl.BlockSpec((rows, _OUT_C), lambda i: (i, 0)),
        out_shape=jax.ShapeDtypeStruct((_N, _OUT_C), jnp.float32),
    )(h, st, gamma2, beta2)


_NC = 2
_NSC = 16
_NW = _NC * _NSC
_QPW = _NS // _NW
_GQ = 8
_NG = _QPW // _GQ


def _segmax_body(h_hbm, nbr_hbm, batch_hbm, fidx_hbm, out_hbm, sb_hbm,
                 idx_v, rows_v, out_v, bidx_v, bval_v, sem):
    c = lax.axis_index("c")
    s = lax.axis_index("s")
    wid = s * _NC + c
    qbase = wid * _QPW

    pltpu.sync_copy(nbr_hbm.at[pl.ds(wid * _NG, _NG)], idx_v)

    pltpu.sync_copy(fidx_hbm.at[pl.ds(qbase, _QPW)], bidx_v)
    pltpu.async_copy(batch_hbm.at[bidx_v], bval_v, sem).wait()
    pltpu.sync_copy(bval_v, sb_hbm.at[pl.ds(qbase, _QPW)])

    def group(g, carry):
        pltpu.async_copy(h_hbm.at[idx_v.at[g]], rows_v, sem).wait()

        def per_q(q, carry2):
            r0 = q * _K
            for cc in range(_OUT_C // 16):
                sl = pl.ds(cc * 16, 16)
                acc = rows_v[r0, sl]
                for r in range(1, _K):
                    acc = jnp.maximum(acc, rows_v[r0 + r, sl])
                out_v[q, sl] = acc
            return carry2

        lax.fori_loop(0, _GQ, per_q, 0)
        pltpu.sync_copy(out_v, out_hbm.at[pl.ds(qbase + g * _GQ, _GQ)])
        return carry

    lax.fori_loop(0, _NG, group, 0)


def _segmax(h, nbr2, batch, fidx):
    mesh = plsc.VectorSubcoreMesh(core_axis_name="c", subcore_axis_name="s")
    f = pl.kernel(
        _segmax_body,
        out_type=[
            jax.ShapeDtypeStruct((_NS, _OUT_C), jnp.float32),
            jax.ShapeDtypeStruct((_NS,), jnp.int32),
        ],
        mesh=mesh,
        scratch_types=[
            pltpu.VMEM((_NG, 128), jnp.int32),
            pltpu.VMEM((_GQ * _K, _OUT_C), jnp.float32),
            pltpu.VMEM((_GQ, _OUT_C), jnp.float32),
            pltpu.VMEM((_QPW,), jnp.int32),
            pltpu.VMEM((_QPW,), jnp.int32),
            pltpu.SemaphoreType.DMA,
        ],
    )
    return f(h, nbr2, batch, fidx)


def kernel(x, pos, batch, W, b, gamma, beta):
    px = pos[:, 0].reshape(128, 128)
    py = pos[:, 1].reshape(128, 128)
    pz = pos[:, 2].reshape(128, 128)
    idxs2, qx2, qy2, qz2 = _fps(px, py, pz)

    sub_pos = jnp.stack(
        [qx2.reshape(-1), qy2.reshape(-1), qz2.reshape(-1)], axis=1)

    zq = jnp.zeros((_NS, 5), jnp.float32)
    qpos = jnp.concatenate(
        [qx2.reshape(-1, 1), qy2.reshape(-1, 1), qz2.reshape(-1, 1), zq],
        axis=1)
    posp = jnp.concatenate([pos, jnp.zeros((_N, 5), jnp.float32)], axis=1)
    cn = jnp.sum(pos * pos, axis=1).reshape(1, _N)
    nbr = _knn(qpos, posp, cn)

    h = _mlp(x, W, b.reshape(1, -1), gamma.reshape(1, -1),
             beta.reshape(1, -1))

    out, sub_batch = _segmax(h, nbr.reshape(_NS * _K // 128, 128), batch,
                             idxs2.reshape(-1))
    return (out, sub_pos, sub_batch)
